# baseline (speedup 1.0000x reference)
import functools

import jax
import jax.numpy as jnp
from jax import lax
from jax.experimental import pallas as pl
from jax.experimental.pallas import tpu as pltpu
from jax.experimental.pallas import tpu_sc as plsc

_BATCH = 4096
_SEQ = 200
_V = 1000000
_D = 32
_NC, _NS = 2, 16
_NW = _NC * _NS
_CB = 512
_BQ = _BATCH // _CB
_BBL = _CB // 128
_UNITS = _SEQ * _BQ
_UPW = _UNITS // _NW

_TILES = (_V + 127) // 128
_VP = _TILES * 128
_CONV_BASE = _TILES // _NW
_CONV_REM = _TILES % _NW


def _build_convert():
    mesh = plsc.VectorSubcoreMesh(core_axis_name="c", subcore_axis_name="s")

    @functools.partial(
        pl.kernel,
        mesh=mesh,
        out_type=jax.ShapeDtypeStruct((_VP * _D,), jnp.float32),
        compiler_params=pltpu.CompilerParams(use_tc_tiling_on_sc=True,
                                             needs_layout_passes=False),
        scratch_types=[
            [pltpu.VMEM((8, 128), jnp.float32) for _ in range(_D // 8)],
            [pltpu.VMEM((8, 128), jnp.float32) for _ in range(_D // 8)],
            pltpu.VMEM((128 * _D,), jnp.float32),
            pltpu.VMEM((128 * _D,), jnp.float32),
            pltpu.SemaphoreType.DMA,
            pltpu.SemaphoreType.DMA,
            pltpu.SemaphoreType.DMA,
            pltpu.SemaphoreType.DMA,
        ],
    )
    def convert_kernel(tt_hbm, lin_hbm, in0, in1, st0, st1, ig0, ig1,
                       ow0, ow1):
        wid = lax.axis_index("s") * _NC + lax.axis_index("c")
        base = wid * _CONV_BASE
        iota = lax.iota(jnp.int32, 16)
        diag = [(s0 + iota) & 7 for s0 in range(8)]
        spat = [iota * _D + diag[s0] for s0 in range(8)]

        inbufs = (in0, in1)
        stbufs = (st0, st1)
        igs = (ig0, ig1)
        ows = (ow0, ow1)

        def in_pairs(ii, p):
            return [(tt_hbm.at[pl.ds(jj * 8, 8), pl.ds(ii * 128, 128)],
                     inbufs[p][jj], igs[p]) for jj in range(_D // 8)]

        def out_pair(ii, p):
            return (stbufs[p], lin_hbm.at[pl.ds(ii * 128 * _D, 128 * _D)],
                    ows[p])

        def start_in(ii, p):
            for pr in in_pairs(ii, p):
                pltpu.async_copy(*pr)

        def wait_in(ii, p):
            for pr in in_pairs(ii, p):
                pltpu.make_async_copy(*pr).wait()

        def shuffle(p):
            stb = stbufs[p]
            for jj in range(_D // 8):
                inb = inbufs[p][jj]
                for s0 in range(8):
                    vs = [plsc.load_gather(inb, [diag[s0], iota + 16 * k])
                          for k in range(8)]
                    for k in range(8):
                        plsc.store_scatter(
                            stb.at[pl.ds(512 * k + jj * 8, 488)],
                            [spat[s0]], vs[k])

        def half(ii, p, g):
            wait_in(ii, p)

            @pl.when(g > 0)
            def _():
                pltpu.make_async_copy(*out_pair(ii - 2, p)).wait()

            shuffle(p)
            pltpu.async_copy(*out_pair(ii, p))
            start_in(ii + 2, p)

        start_in(base, 0)
        start_in(base + 1, 1)

        def body(g, carry):
            half(base + 2 * g, 0, g)
            half(base + 2 * g + 1, 1, g)
            return carry

        npairs = _CONV_BASE // 2
        lax.fori_loop(0, npairs, body, 0)

        wait_in(base + _CONV_BASE, 0)
        wait_in(base + _CONV_BASE + 1, 1)
        pltpu.make_async_copy(*out_pair(base + _CONV_BASE - 2, 0)).wait()
        pltpu.make_async_copy(*out_pair(base + _CONV_BASE - 1, 1)).wait()

        @pl.when(wid < _CONV_REM)
        def _():
            ii = _NW * _CONV_BASE + wid
            start_in(ii, 0)
            wait_in(ii, 0)
            shuffle(0)
            pltpu.async_copy(*out_pair(ii, 0))
            pltpu.make_async_copy(*out_pair(ii, 0)).wait()

    return convert_kernel


def _build():
    mesh = plsc.VectorSubcoreMesh(core_axis_name="c", subcore_axis_name="s")

    @functools.partial(
        pl.kernel,
        mesh=mesh,
        out_type=jax.ShapeDtypeStruct((_SEQ, _D // 8, _BATCH * 8),
                                      jnp.float32),
        compiler_params=pltpu.CompilerParams(use_tc_tiling_on_sc=False,
                                             needs_layout_passes=False),
        scratch_types=[
            pltpu.VMEM((_CB,), jnp.int32),
            pltpu.VMEM((_CB,), jnp.int32),
            pltpu.VMEM((_CB, _D), jnp.float32),
            pltpu.VMEM((_CB, _D), jnp.float32),
            pltpu.VMEM((_CB * _D,), jnp.float32),
            pltpu.VMEM((_CB * _D,), jnp.float32),
            pltpu.SemaphoreType.DMA,
            pltpu.SemaphoreType.DMA,
            pltpu.SemaphoreType.DMA,
            pltpu.SemaphoreType.DMA,
            pltpu.SemaphoreType.DMA,
            pltpu.SemaphoreType.DMA,
        ],
    )
    def gather_kernel(idx_hbm, table_hbm, z_hbm, ib0, ib1, gb0, gb1, sb0, sb1,
                      is0, is1, gs0, gs1, ws0, ws1):
        wid = lax.axis_index("s") * _NC + lax.axis_index("c")
        u0 = wid * _UPW
        iota = lax.iota(jnp.int32, 16)
        diag = [(s0 + iota) & 7 for s0 in range(8)]
        dpat = [diag[s0] * 128 + iota for s0 in range(8)]

        ibufs = (ib0, ib1)
        gbufs = (gb0, gb1)
        sbufs = (sb0, sb1)
        isems = (is0, is1)
        gsems = (gs0, gs1)
        wsems = (ws0, ws1)

        def idx_src(u):
            l = u // _BQ
            bq = u % _BQ
            return idx_hbm.at[pl.ds(l * _BATCH + bq * _CB, _CB)]

        def start_idx(u, p):
            pltpu.async_copy(idx_src(u), ibufs[p], isems[p])

        def wait_idx(u, p):
            pltpu.make_async_copy(idx_src(u), ibufs[p], isems[p]).wait()

        def start_gather(p):
            pltpu.async_copy(table_hbm.at[ibufs[p]], gbufs[p], gsems[p])

        def wait_gather(p):
            pltpu.make_async_copy(table_hbm.at[ibufs[p]], gbufs[p],
                                  gsems[p]).wait()

        def wb_pairs(u, p):
            l = u // _BQ
            bq = u % _BQ
            return [(sbufs[p].at[pl.ds(jj * _CB * 8, _CB * 8)],
                     z_hbm.at[l, jj, pl.ds(bq * _CB * 8, _CB * 8)])
                    for jj in range(_D // 8)]

        def start_wb(u, p):
            for src, dst in wb_pairs(u, p):
                pltpu.async_copy(src, dst, wsems[p])

        def wait_wb(u, p):
            for src, dst in wb_pairs(u, p):
                pltpu.make_async_copy(src, dst, wsems[p]).wait()

        def shuffle(p):
            gb = gbufs[p]
            sb = sbufs[p]

            def body(t, carry):
                jj = t // _BBL
                bbl = t % _BBL
                for s0 in range(0, 8, 2):
                    colv0 = diag[s0] + jj * 8
                    colv1 = diag[s0 + 1] + jj * 8
                    vs = ([plsc.load_gather(
                               gb, [iota + (bbl * 128 + 16 * k), colv0])
                           for k in range(8)] +
                          [plsc.load_gather(
                               gb, [iota + (bbl * 128 + 16 * k), colv1])
                           for k in range(8)])
                    for j, s in enumerate((s0, s0 + 1)):
                        for k in range(8):
                            dstv = dpat[s] + (jj * 4096 + bbl * 1024
                                              + 16 * k)
                            plsc.store_scatter(sb, [dstv], vs[8 * j + k])
                return carry

            lax.fori_loop(0, (_D // 8) * _BBL, body, 0)

        start_idx(u0, 0)
        wait_idx(u0, 0)
        start_gather(0)
        start_idx(u0 + 1, 1)

        def pair_body(g, carry):
            u = u0 + 2 * g
            wait_idx(u + 1, 1)
            start_gather(1)
            wait_gather(0)
            start_idx(u + 2, 0)

            @pl.when(g > 0)
            def _():
                wait_wb(u - 2, 0)

            shuffle(0)
            start_wb(u, 0)
            wait_idx(u + 2, 0)
            start_gather(0)
            wait_gather(1)
            start_idx(u + 3, 1)

            @pl.when(g > 0)
            def _():
                wait_wb(u - 1, 1)

            shuffle(1)
            start_wb(u + 1, 1)
            return carry

        lax.fori_loop(0, _UPW // 2 - 1, pair_body, 0)

        u = u0 + _UPW - 2
        wait_idx(u + 1, 1)
        start_gather(1)
        wait_gather(0)
        wait_wb(u - 2, 0)
        shuffle(0)
        start_wb(u, 0)
        wait_gather(1)
        wait_wb(u - 1, 1)
        shuffle(1)
        start_wb(u + 1, 1)
        wait_wb(u, 0)
        wait_wb(u + 1, 1)

    return gather_kernel


_CONVERT = _build_convert()
_GATHER = _build()


def kernel(tokens, table):
    idx = tokens.T.reshape(-1).astype(jnp.int32)
    lin = _CONVERT(table.T)
    z = _GATHER(idx, lin.reshape(_VP, _D))
    z = z.reshape(_SEQ, _D // 8, _BATCH // 128, 8, 128)
    return z.transpose(2, 4, 0, 1, 3).reshape(_BATCH, _SEQ, _D)


# speedup vs baseline: 1.0044x; 1.0044x over previous
import functools

import jax
import jax.numpy as jnp
from jax import lax
from jax.experimental import pallas as pl
from jax.experimental.pallas import tpu as pltpu
from jax.experimental.pallas import tpu_sc as plsc

_BATCH = 4096
_SEQ = 200
_V = 1000000
_D = 32
_NC, _NS = 2, 16
_NW = _NC * _NS
_CB = 512
_BQ = _BATCH // _CB
_BBL = _CB // 128
_UNITS = _SEQ * _BQ
_UPW = _UNITS // _NW

_TILES = (_V + 127) // 128
_VP = _TILES * 128
_CONV_BASE = _TILES // _NW
_CONV_REM = _TILES % _NW


def _build_convert():
    mesh = plsc.VectorSubcoreMesh(core_axis_name="c", subcore_axis_name="s")

    @functools.partial(
        pl.kernel,
        mesh=mesh,
        out_type=jax.ShapeDtypeStruct((_VP * _D,), jnp.float32),
        compiler_params=pltpu.CompilerParams(use_tc_tiling_on_sc=True,
                                             needs_layout_passes=False),
        scratch_types=[
            pltpu.VMEM((_D, 128), jnp.float32),
            pltpu.VMEM((_D, 128), jnp.float32),
            pltpu.VMEM((128 * _D,), jnp.float32),
            pltpu.VMEM((128 * _D,), jnp.float32),
            pltpu.SemaphoreType.DMA,
            pltpu.SemaphoreType.DMA,
            pltpu.SemaphoreType.DMA,
            pltpu.SemaphoreType.DMA,
        ],
    )
    def convert_kernel(tt_hbm, lin_hbm, in0, in1, st0, st1, ig0, ig1,
                       ow0, ow1):
        wid = lax.axis_index("s") * _NC + lax.axis_index("c")
        base = wid * _CONV_BASE
        iota = lax.iota(jnp.int32, 16)
        diag = [(s0 + iota) & 7 for s0 in range(8)]
        spat = [iota * _D + diag[s0] for s0 in range(8)]

        inbufs = (in0, in1)
        stbufs = (st0, st1)
        igs = (ig0, ig1)
        ows = (ow0, ow1)

        def in_pair(ii, p):
            return (tt_hbm.at[:, pl.ds(ii * 128, 128)], inbufs[p], igs[p])

        def out_pair(ii, p):
            return (stbufs[p], lin_hbm.at[pl.ds(ii * 128 * _D, 128 * _D)],
                    ows[p])

        def start_in(ii, p):
            pltpu.async_copy(*in_pair(ii, p))

        def wait_in(ii, p):
            pltpu.make_async_copy(*in_pair(ii, p)).wait()

        def shuffle(p):
            stb = stbufs[p]
            for jj in range(_D // 8):
                inb = inbufs[p].at[pl.ds(jj * 8, 8)]
                for s0 in range(8):
                    vs = [plsc.load_gather(inb, [diag[s0], iota + 16 * k])
                          for k in range(8)]
                    for k in range(8):
                        plsc.store_scatter(
                            stb.at[pl.ds(512 * k + jj * 8, 488)],
                            [spat[s0]], vs[k])

        def half(ii, p, g):
            wait_in(ii, p)

            @pl.when(g > 0)
            def _():
                pltpu.make_async_copy(*out_pair(ii - 2, p)).wait()

            shuffle(p)
            pltpu.async_copy(*out_pair(ii, p))
            start_in(ii + 2, p)

        start_in(base, 0)
        start_in(base + 1, 1)

        def body(g, carry):
            half(base + 2 * g, 0, g)
            half(base + 2 * g + 1, 1, g)
            return carry

        npairs = _CONV_BASE // 2
        lax.fori_loop(0, npairs, body, 0)

        wait_in(base + _CONV_BASE, 0)
        wait_in(base + _CONV_BASE + 1, 1)
        pltpu.make_async_copy(*out_pair(base + _CONV_BASE - 2, 0)).wait()
        pltpu.make_async_copy(*out_pair(base + _CONV_BASE - 1, 1)).wait()

        @pl.when(wid < _CONV_REM)
        def _():
            ii = _NW * _CONV_BASE + wid
            start_in(ii, 0)
            wait_in(ii, 0)
            shuffle(0)
            pltpu.async_copy(*out_pair(ii, 0))
            pltpu.make_async_copy(*out_pair(ii, 0)).wait()

    return convert_kernel


def _build():
    mesh = plsc.VectorSubcoreMesh(core_axis_name="c", subcore_axis_name="s")

    @functools.partial(
        pl.kernel,
        mesh=mesh,
        out_type=jax.ShapeDtypeStruct((_SEQ, _D // 8, _BATCH * 8),
                                      jnp.float32),
        compiler_params=pltpu.CompilerParams(use_tc_tiling_on_sc=False,
                                             needs_layout_passes=False),
        scratch_types=[
            pltpu.VMEM((_CB,), jnp.int32),
            pltpu.VMEM((_CB,), jnp.int32),
            pltpu.VMEM((_CB, _D), jnp.float32),
            pltpu.VMEM((_CB, _D), jnp.float32),
            pltpu.VMEM((_CB * _D,), jnp.float32),
            pltpu.VMEM((_CB * _D,), jnp.float32),
            pltpu.SemaphoreType.DMA,
            pltpu.SemaphoreType.DMA,
            pltpu.SemaphoreType.DMA,
            pltpu.SemaphoreType.DMA,
            pltpu.SemaphoreType.DMA,
            pltpu.SemaphoreType.DMA,
        ],
    )
    def gather_kernel(idx_hbm, table_hbm, z_hbm, ib0, ib1, gb0, gb1, sb0, sb1,
                      is0, is1, gs0, gs1, ws0, ws1):
        wid = lax.axis_index("s") * _NC + lax.axis_index("c")
        u0 = wid * _UPW
        iota = lax.iota(jnp.int32, 16)
        diag = [(s0 + iota) & 7 for s0 in range(8)]
        dpat = [diag[s0] * 128 + iota for s0 in range(8)]

        ibufs = (ib0, ib1)
        gbufs = (gb0, gb1)
        sbufs = (sb0, sb1)
        isems = (is0, is1)
        gsems = (gs0, gs1)
        wsems = (ws0, ws1)

        def idx_src(u):
            l = u // _BQ
            bq = u % _BQ
            return idx_hbm.at[pl.ds(l * _BATCH + bq * _CB, _CB)]

        def start_idx(u, p):
            pltpu.async_copy(idx_src(u), ibufs[p], isems[p])

        def wait_idx(u, p):
            pltpu.make_async_copy(idx_src(u), ibufs[p], isems[p]).wait()

        def start_gather(p):
            pltpu.async_copy(table_hbm.at[ibufs[p]], gbufs[p], gsems[p])

        def wait_gather(p):
            pltpu.make_async_copy(table_hbm.at[ibufs[p]], gbufs[p],
                                  gsems[p]).wait()

        def wb_pairs(u, p):
            l = u // _BQ
            bq = u % _BQ
            return [(sbufs[p].at[pl.ds(jj * _CB * 8, _CB * 8)],
                     z_hbm.at[l, jj, pl.ds(bq * _CB * 8, _CB * 8)])
                    for jj in range(_D // 8)]

        def start_wb(u, p):
            for src, dst in wb_pairs(u, p):
                pltpu.async_copy(src, dst, wsems[p])

        def wait_wb(u, p):
            for src, dst in wb_pairs(u, p):
                pltpu.make_async_copy(src, dst, wsems[p]).wait()

        def shuffle(p):
            gb = gbufs[p]
            sb = sbufs[p]

            def body(t, carry):
                jj = t // _BBL
                bbl = t % _BBL
                for s0 in range(8):
                    colv = diag[s0] + jj * 8
                    vs = [plsc.load_gather(
                              gb, [iota + (bbl * 128 + 16 * k), colv])
                          for k in range(8)]
                    for k in range(8):
                        dstv = dpat[s0] + (jj * 4096 + bbl * 1024 + 16 * k)
                        plsc.store_scatter(sb, [dstv], vs[k])
                return carry

            lax.fori_loop(0, (_D // 8) * _BBL, body, 0)

        start_idx(u0, 0)
        wait_idx(u0, 0)
        start_gather(0)
        start_idx(u0 + 1, 1)

        def pair_body(g, carry):
            u = u0 + 2 * g
            wait_idx(u + 1, 1)
            start_gather(1)
            wait_gather(0)
            start_idx(u + 2, 0)

            @pl.when(g > 0)
            def _():
                wait_wb(u - 2, 0)

            shuffle(0)
            start_wb(u, 0)
            wait_idx(u + 2, 0)
            start_gather(0)
            wait_gather(1)
            start_idx(u + 3, 1)

            @pl.when(g > 0)
            def _():
                wait_wb(u - 1, 1)

            shuffle(1)
            start_wb(u + 1, 1)
            return carry

        lax.fori_loop(0, _UPW // 2 - 1, pair_body, 0)

        u = u0 + _UPW - 2
        wait_idx(u + 1, 1)
        start_gather(1)
        wait_gather(0)
        wait_wb(u - 2, 0)
        shuffle(0)
        start_wb(u, 0)
        wait_gather(1)
        wait_wb(u - 1, 1)
        shuffle(1)
        start_wb(u + 1, 1)
        wait_wb(u, 0)
        wait_wb(u + 1, 1)

    return gather_kernel


_CONVERT = _build_convert()
_GATHER = _build()


def kernel(tokens, table):
    idx = tokens.T.reshape(-1).astype(jnp.int32)
    lin = _CONVERT(table.T)
    z = _GATHER(idx, lin.reshape(_VP, _D))
    z = z.reshape(_SEQ, _D // 8, _BATCH // 128, 8, 128)
    return z.transpose(2, 4, 0, 1, 3).reshape(_BATCH, _SEQ, _D)


# speedup vs baseline: 1.0068x; 1.0024x over previous
import functools

import jax
import jax.numpy as jnp
from jax import lax
from jax.experimental import pallas as pl
from jax.experimental.pallas import tpu as pltpu
from jax.experimental.pallas import tpu_sc as plsc

_BATCH = 4096
_SEQ = 200
_V = 1000000
_D = 32
_NC, _NS = 2, 16
_NW = _NC * _NS
_CB = 256
_BQ = _BATCH // _CB
_BBL = _CB // 128
_UNITS = _SEQ * _BQ
_UPW = _UNITS // _NW

_TILES = (_V + 127) // 128
_VP = _TILES * 128
_CONV_BASE = _TILES // _NW
_CONV_REM = _TILES % _NW


def _build_convert():
    mesh = plsc.VectorSubcoreMesh(core_axis_name="c", subcore_axis_name="s")

    @functools.partial(
        pl.kernel,
        mesh=mesh,
        out_type=jax.ShapeDtypeStruct((_VP * _D,), jnp.float32),
        compiler_params=pltpu.CompilerParams(use_tc_tiling_on_sc=True,
                                             needs_layout_passes=False),
        scratch_types=[
            pltpu.VMEM((_D, 128), jnp.float32),
            pltpu.VMEM((_D, 128), jnp.float32),
            pltpu.VMEM((128 * _D,), jnp.float32),
            pltpu.VMEM((128 * _D,), jnp.float32),
            pltpu.SemaphoreType.DMA,
            pltpu.SemaphoreType.DMA,
            pltpu.SemaphoreType.DMA,
            pltpu.SemaphoreType.DMA,
        ],
    )
    def convert_kernel(tt_hbm, lin_hbm, in0, in1, st0, st1, ig0, ig1,
                       ow0, ow1):
        wid = lax.axis_index("s") * _NC + lax.axis_index("c")
        base = wid * _CONV_BASE
        iota = lax.iota(jnp.int32, 16)
        diag = [(s0 + iota) & 7 for s0 in range(8)]
        spat = [iota * _D + diag[s0] for s0 in range(8)]

        inbufs = (in0, in1)
        stbufs = (st0, st1)
        igs = (ig0, ig1)
        ows = (ow0, ow1)

        def in_pair(ii, p):
            return (tt_hbm.at[:, pl.ds(ii * 128, 128)], inbufs[p], igs[p])

        def out_pair(ii, p):
            return (stbufs[p], lin_hbm.at[pl.ds(ii * 128 * _D, 128 * _D)],
                    ows[p])

        def start_in(ii, p):
            pltpu.async_copy(*in_pair(ii, p))

        def wait_in(ii, p):
            pltpu.make_async_copy(*in_pair(ii, p)).wait()

        def shuffle(p):
            stb = stbufs[p]
            for jj in range(_D // 8):
                inb = inbufs[p].at[pl.ds(jj * 8, 8)]
                for s0 in range(8):
                    vs = [plsc.load_gather(inb, [diag[s0], iota + 16 * k])
                          for k in range(8)]
                    for k in range(8):
                        plsc.store_scatter(
                            stb.at[pl.ds(512 * k + jj * 8, 488)],
                            [spat[s0]], vs[k])

        def half(ii, p, g):
            wait_in(ii, p)

            @pl.when(g > 0)
            def _():
                pltpu.make_async_copy(*out_pair(ii - 2, p)).wait()

            shuffle(p)
            pltpu.async_copy(*out_pair(ii, p))
            start_in(ii + 2, p)

        start_in(base, 0)
        start_in(base + 1, 1)

        def body(g, carry):
            half(base + 2 * g, 0, g)
            half(base + 2 * g + 1, 1, g)
            return carry

        npairs = _CONV_BASE // 2
        lax.fori_loop(0, npairs, body, 0)

        wait_in(base + _CONV_BASE, 0)
        wait_in(base + _CONV_BASE + 1, 1)
        pltpu.make_async_copy(*out_pair(base + _CONV_BASE - 2, 0)).wait()
        pltpu.make_async_copy(*out_pair(base + _CONV_BASE - 1, 1)).wait()

        @pl.when(wid < _CONV_REM)
        def _():
            ii = _NW * _CONV_BASE + wid
            start_in(ii, 0)
            wait_in(ii, 0)
            shuffle(0)
            pltpu.async_copy(*out_pair(ii, 0))
            pltpu.make_async_copy(*out_pair(ii, 0)).wait()

    return convert_kernel


def _build():
    mesh = plsc.VectorSubcoreMesh(core_axis_name="c", subcore_axis_name="s")

    @functools.partial(
        pl.kernel,
        mesh=mesh,
        out_type=jax.ShapeDtypeStruct((_SEQ, _D // 8, _BATCH * 8),
                                      jnp.float32),
        compiler_params=pltpu.CompilerParams(use_tc_tiling_on_sc=False,
                                             needs_layout_passes=False),
        scratch_types=(
            [pltpu.VMEM((_CB,), jnp.int32) for _ in range(4)]
            + [pltpu.VMEM((_CB, _D), jnp.float32) for _ in range(4)]
            + [pltpu.VMEM((_CB * _D,), jnp.float32) for _ in range(4)]
            + [pltpu.SemaphoreType.DMA for _ in range(12)]
        ),
    )
    def gather_kernel(idx_hbm, table_hbm, z_hbm,
                      ib0, ib1, ib2, ib3, gb0, gb1, gb2, gb3,
                      sb0, sb1, sb2, sb3,
                      is0, is1, is2, is3, gs0, gs1, gs2, gs3,
                      ws0, ws1, ws2, ws3):
        wid = lax.axis_index("s") * _NC + lax.axis_index("c")
        u0 = wid * _UPW
        iota = lax.iota(jnp.int32, 16)
        diag = [(s0 + iota) & 7 for s0 in range(8)]
        dpat = [diag[s0] * 128 + iota for s0 in range(8)]

        ibufs = (ib0, ib1, ib2, ib3)
        gbufs = (gb0, gb1, gb2, gb3)
        sbufs = (sb0, sb1, sb2, sb3)
        isems = (is0, is1, is2, is3)
        gsems = (gs0, gs1, gs2, gs3)
        wsems = (ws0, ws1, ws2, ws3)

        def idx_src(u):
            l = u // _BQ
            bq = u % _BQ
            return idx_hbm.at[pl.ds(l * _BATCH + bq * _CB, _CB)]

        def start_idx(u, p):
            pltpu.async_copy(idx_src(u), ibufs[p], isems[p])

        def wait_idx(u, p):
            pltpu.make_async_copy(idx_src(u), ibufs[p], isems[p]).wait()

        def start_gather(p):
            pltpu.async_copy(table_hbm.at[ibufs[p]], gbufs[p], gsems[p])

        def wait_gather(p):
            pltpu.make_async_copy(table_hbm.at[ibufs[p]], gbufs[p],
                                  gsems[p]).wait()

        def wb_pairs(u, p):
            l = u // _BQ
            bq = u % _BQ
            return [(sbufs[p].at[pl.ds(jj * _CB * 8, _CB * 8)],
                     z_hbm.at[l, jj, pl.ds(bq * _CB * 8, _CB * 8)])
                    for jj in range(_D // 8)]

        def start_wb(u, p):
            for src, dst in wb_pairs(u, p):
                pltpu.async_copy(src, dst, wsems[p])

        def wait_wb(u, p):
            for src, dst in wb_pairs(u, p):
                pltpu.make_async_copy(src, dst, wsems[p]).wait()

        def shuffle(p):
            gb = gbufs[p]
            sb = sbufs[p]

            def body(t, carry):
                jj = t // _BBL
                bbl = t % _BBL
                for s0 in range(8):
                    colv = diag[s0] + jj * 8
                    vs = [plsc.load_gather(
                              gb, [iota + (bbl * 128 + 16 * k), colv])
                          for k in range(8)]
                    for k in range(8):
                        dstv = dpat[s0] + (jj * _CB * 8 + bbl * 1024
                                           + 16 * k)
                        plsc.store_scatter(sb, [dstv], vs[k])
                return carry

            lax.fori_loop(0, (_D // 8) * _BBL, body, 0)

        start_idx(u0, 0)
        start_idx(u0 + 1, 1)
        wait_idx(u0, 0)
        start_gather(0)
        start_idx(u0 + 2, 2)
        wait_idx(u0 + 1, 1)
        start_gather(1)
        start_idx(u0 + 3, 3)

        def quad_body(g, carry):
            for q in range(4):
                u = u0 + 4 * g + q
                wait_idx(u + 2, (q + 2) % 4)
                start_gather((q + 2) % 4)
                wait_gather(q)
                start_idx(u + 4, q)

                @pl.when(g > 0)
                def _():
                    wait_wb(u - 4, q)

                shuffle(q)
                start_wb(u, q)
            return carry

        lax.fori_loop(0, _UPW // 4 - 1, quad_body, 0)

        ue = u0 + _UPW - 4
        for q in range(4):
            u = ue + q
            if q < 2:
                wait_idx(u + 2, (q + 2) % 4)
                start_gather((q + 2) % 4)
            wait_gather(q)
            wait_wb(u - 4, q)
            shuffle(q)
            start_wb(u, q)
        for q in range(4):
            wait_wb(ue + q, q)

    return gather_kernel


_CONVERT = _build_convert()
_GATHER = _build()


def kernel(tokens, table):
    idx = tokens.T.reshape(-1).astype(jnp.int32)
    lin = _CONVERT(table.T)
    z = _GATHER(idx, lin.reshape(_VP, _D))
    z = z.reshape(_SEQ, _D // 8, _BATCH // 128, 8, 128)
    return z.transpose(2, 4, 0, 1, 3).reshape(_BATCH, _SEQ, _D)
